# baseline (speedup 1.0000x reference)
import functools

import jax
import jax.numpy as jnp
from jax import lax
from jax.experimental import pallas as pl
from jax.experimental.pallas import tpu as pltpu
from jax.experimental.pallas import tpu_sc as plsc

PAD = 0
SMOOTHING = 0.1
CONFIDENCE = 1.0 - SMOOTHING

N_ROWS = 1024
N_COLS = 100000
LANES = 16
COLS_LG = N_COLS // LANES

NC, NS = 2, 16
NW = NC * NS
BPW = N_ROWS // NW
CH = BPW // LANES

TROW, TCOL = 8, 128


def _sc_gather_body(x_hbm, tgt_hbm, out_hbm, tgt_v, tiles_v, out_v, sem):
    wid = lax.axis_index("s") * NC + lax.axis_index("c")
    base = wid * BPW
    pltpu.sync_copy(tgt_hbm.at[pl.ds(base, BPW)], tgt_v)
    descs = []
    for c in range(CH):
        tv = tgt_v[pl.ds(c * LANES, LANES)]
        for j in range(LANES):
            k = c * LANES + j
            col128 = pl.multiple_of((tv[j] >> 7) << 7, TCOL)
            row8 = pl.multiple_of(base + (k & ~(TROW - 1)), TROW)
            d = pltpu.make_async_copy(
                x_hbm.at[pl.ds(row8, TROW), pl.ds(col128, TCOL)],
                tiles_v.at[k],
                sem,
            )
            d.start()
            descs.append(d)
    for d in descs:
        d.wait()
    for k in range(BPW):
        r = k % TROW
        for l in range(TCOL // LANES):
            out_v[k, pl.ds(l * LANES, LANES)] = tiles_v[k, r, pl.ds(l * LANES, LANES)]
    pltpu.sync_copy(out_v, out_hbm.at[pl.ds(base, BPW)])


@functools.cache
def _sc_gather():
    mesh = plsc.VectorSubcoreMesh(
        core_axis_name="c", subcore_axis_name="s", num_cores=NC, num_subcores=NS
    )
    return pl.kernel(
        _sc_gather_body,
        out_type=jax.ShapeDtypeStruct((N_ROWS, TCOL), jnp.float32),
        mesh=mesh,
        scratch_types=[
            pltpu.VMEM((BPW,), jnp.int32),
            pltpu.VMEM((BPW, TROW, TCOL), jnp.float32),
            pltpu.VMEM((BPW, TCOL), jnp.float32),
            pltpu.SemaphoreType.DMA,
        ],
    )


EPS = SMOOTHING / (N_COLS - 2)

NBUF = 8
BRM = 16
GSTEPS = N_ROWS // (NBUF * BRM)


def _blk_contrib(blk, t, xtg):
    s = jnp.sum(blk, axis=1, keepdims=True)
    lanes = lax.broadcasted_iota(jnp.int32, (BRM, TCOL), 1)
    xt = jnp.sum(
        jnp.where(lanes == (t & (TCOL - 1)), xtg, 0.0), axis=1, keepdims=True
    )
    row = EPS * (s - blk[:, 0:1] - xt) + CONFIDENCE * xt
    row = jnp.where(t != PAD, row, 0.0)
    return jnp.sum(row)


def _tc_body(t_ref, xtg_ref, x_hbm, out_ref, acc_ref, *bufs_sems):
    bufs, sems = bufs_sems[:NBUF], bufs_sems[NBUF:]
    g = pl.program_id(0)

    @pl.when(g == 0)
    def _():
        acc_ref[0] = 0.0
        for k in range(NBUF):
            pltpu.make_async_copy(
                x_hbm.at[pl.ds(k * BRM, BRM)], bufs[k], sems[k]
            ).start()

    part = jnp.float32(0.0)
    for k in range(NBUF):
        pltpu.make_async_copy(
            x_hbm.at[pl.ds(0, BRM)], bufs[k], sems[k]
        ).wait()
        row0 = (g * NBUF + k) * BRM
        t = t_ref[pl.ds(row0, BRM), :]
        xtg = xtg_ref[pl.ds(row0, BRM), :]
        part += _blk_contrib(bufs[k][...], t, xtg)

        @pl.when(g + 1 < GSTEPS)
        def _():
            nxt = ((g + 1) * NBUF + k) * BRM
            pltpu.make_async_copy(
                x_hbm.at[pl.ds(nxt, BRM)], bufs[k], sems[k]
            ).start()

    acc_ref[0] += part

    @pl.when(g == GSTEPS - 1)
    def _():
        out_ref[0, 0] = -acc_ref[0] / N_ROWS


def kernel(x, target):
    target = target.astype(jnp.int32)
    xtg = _sc_gather()(x, target)
    loss = pl.pallas_call(
        _tc_body,
        grid=(GSTEPS,),
        in_specs=[
            pl.BlockSpec((N_ROWS, 1), lambda g: (0, 0)),
            pl.BlockSpec((N_ROWS, TCOL), lambda g: (0, 0)),
            pl.BlockSpec(memory_space=pl.ANY),
        ],
        out_specs=pl.BlockSpec(memory_space=pltpu.SMEM),
        out_shape=jax.ShapeDtypeStruct((1, 1), jnp.float32),
        scratch_shapes=(
            [pltpu.SMEM((1,), jnp.float32)]
            + [pltpu.VMEM((BRM, N_COLS), jnp.float32) for _ in range(NBUF)]
            + [pltpu.SemaphoreType.DMA for _ in range(NBUF)]
        ),
    )(target.reshape(N_ROWS, 1), xtg, x)
    return loss[0, 0]
